# Initial kernel scaffold; baseline (speedup 1.0000x reference)
import jax
import jax.numpy as jnp
from jax.experimental import pallas as pl


def kernel(total_energy, species_index, shifts, total_shift):
    raise NotImplementedError("write your pallas kernel here")


# speedup vs baseline: 1.2135x; 1.2135x over previous
import jax
import jax.numpy as jnp
from jax import lax
from jax.experimental import pallas as pl
from jax.experimental.pallas import tpu as pltpu
from jax.experimental.pallas import tpu_sc as plsc

N_SP = 64
N_AT = 100000
NSUB = 16
LANES = 16
CHUNK = 6272
PAD_N = NSUB * CHUNK
STEPS = CHUNK // LANES
TBL = 128


def _sc_body(idx_hbm, shifts_hbm, part_hbm, idx_v, tbl_v, vec_v):
    sid = lax.axis_index("s")
    base = sid * CHUNK
    pltpu.sync_copy(shifts_hbm, tbl_v)
    pltpu.sync_copy(idx_hbm.at[pl.ds(base, CHUNK)], idx_v)

    def step(j, acc):
        idx16 = idx_v[pl.ds(j * LANES, LANES)]
        return acc + plsc.load_gather(tbl_v, [idx16])

    acc = lax.fori_loop(0, STEPS, step, jnp.zeros((LANES,), jnp.float32),
                        unroll=8)
    vec_v[...] = acc
    pltpu.sync_copy(vec_v, part_hbm.at[sid])


def _tc_finish(part_ref, te_ref, ts_ref, out_ref):
    s = jnp.sum(part_ref[...])
    out_ref[...] = te_ref[...] + ts_ref[...] + s


@jax.jit
def _shift_sum(idx_pad, shifts_pad, te_row, ts_row):
    mesh = plsc.VectorSubcoreMesh(core_axis_name="c", subcore_axis_name="s",
                                  num_cores=1)
    partials = pl.kernel(
        _sc_body,
        out_type=jax.ShapeDtypeStruct((NSUB, LANES), jnp.float32),
        mesh=mesh,
        compiler_params=pltpu.CompilerParams(needs_layout_passes=False),
        scratch_types=[
            pltpu.VMEM((CHUNK,), jnp.int32),
            pltpu.VMEM((TBL,), jnp.float32),
            pltpu.VMEM((LANES,), jnp.float32),
        ],
    )(idx_pad, shifts_pad)
    return pl.pallas_call(
        _tc_finish,
        out_shape=jax.ShapeDtypeStruct((1, LANES), jnp.float32),
    )(partials, te_row, ts_row)


def kernel(total_energy, species_index, shifts, total_shift):
    idx = species_index.astype(jnp.int32)
    idx_pad = jnp.full((PAD_N,), N_SP, jnp.int32).at[:N_AT].set(idx)
    shifts_pad = jnp.zeros((TBL,), jnp.float32).at[:N_SP].set(shifts)
    te_row = total_energy.reshape(1, LANES)
    ts_row = jnp.broadcast_to(total_shift.astype(jnp.float32), (1, LANES))
    out = _shift_sum(idx_pad, shifts_pad, te_row, ts_row)
    return out.reshape(total_energy.shape)
